# baseline (speedup 1.0000x reference)
import jax
import jax.numpy as jnp
from jax import lax
from jax.experimental import pallas as pl
from jax.experimental.pallas import tpu as pltpu
from jax.experimental.pallas import tpu_sc as plsc

N_NODES = 50000
D = 128
N_LABELS = 4
E_PER_LABEL = 150000

NS = 16
E_TOT = N_LABELS * E_PER_LABEL
BLK = 1024
NBLK = 40
EPT = NBLK * BLK
E_PAD = NS * EPT
BATCH = 128
RING = 4
LAG = 2
NCH = 8
CHUNK = 6256
ROWS_PER_TILE = 392
ACC_ROWS = 6272
SHIFT = 13
LMASK = (1 << SHIFT) - 1
JUNK_LDST = 6260
PAD_SRC = 0
CAP_P = 8192
TAB_ROWS = (N_LABELS + 1) * N_NODES
RB = 2000


def _tc_body(x_ref, w_ref, b_ref, o_ref):
    j = pl.program_id(1)
    y = jnp.dot(x_ref[...], w_ref[0], preferred_element_type=jnp.float32)
    o_ref[...] = y + jnp.where(j == 0, b_ref[...], jnp.zeros_like(b_ref[...]))


def _tc_table(x, w_cat, b2d):
    n_i = N_NODES // RB
    return pl.pallas_call(
        _tc_body,
        grid=(n_i, N_LABELS + 1),
        in_specs=[
            pl.BlockSpec((RB, D), lambda i, j: (i, 0)),
            pl.BlockSpec((1, D, D), lambda i, j: (j, 0, 0)),
            pl.BlockSpec((1, D), lambda i, j: (0, 0)),
        ],
        out_specs=pl.BlockSpec((RB, D), lambda i, j: (j * (N_NODES // RB) + i, 0)),
        out_shape=jax.ShapeDtypeStruct((TAB_ROWS, D), jnp.float32),
    )(x, w_cat, b2d)


def _sc_body(tab, gsrc, dst, out,
             acc, pairs, gin0, gin1, din0, din1, gq, dq, rows, sin, sg, ss):
    gin = (gin0, gin1)
    din = (din0, din1)
    cid = lax.axis_index("c")
    tid = lax.axis_index("s")
    slab = tid * EPT
    r0 = tid * ROWS_PER_TILE
    junk16 = jnp.full((16,), JUNK_LDST, jnp.int32)

    def wait_gather(s):
        pltpu.make_async_copy(tab.at[gq.at[s]], rows.at[s], sg.at[s]).wait()

    def wait_scatter(s):
        pltpu.make_async_copy(rows.at[s], acc.at[dq.at[s]], ss.at[s]).wait()

    def fire(b, carry):
        s = lax.rem(b, RING)

        @pl.when(b >= RING)
        def _():
            wait_scatter(s)
        base = (b * BATCH) & (CAP_P - 1)
        for k in range(BATCH // 16):
            p16 = pairs[pl.ds(base + k * 16, 16)]
            gq[s, pl.ds(k * 16, 16)] = (
                lax.shift_right_logical(p16, SHIFT) + N_NODES)
            dq[s, pl.ds(k * 16, 16)] = p16 & LMASK
        pltpu.async_copy(tab.at[gq.at[s]], rows.at[s], sg.at[s])

        @pl.when(b >= LAG)
        def _():
            s2 = lax.rem(b - LAG, RING)
            wait_gather(s2)
            pltpu.async_copy(rows.at[s2], acc.at[dq.at[s2]], ss.at[s2],
                             add=True)
        return carry

    def chase(t, carry):
        s2 = lax.rem(t, RING)
        wait_gather(s2)
        pltpu.async_copy(rows.at[s2], acc.at[dq.at[s2]], ss.at[s2], add=True)
        return carry

    def drain_scatter(t, carry):
        wait_scatter(lax.rem(t, RING))
        return carry

    def do_chunk(j):
        lo = (cid * (NCH // 2) + j) * CHUNK

        init_desc = pltpu.make_async_copy(
            tab.at[pl.ds(lo + r0, ROWS_PER_TILE)],
            acc.at[pl.ds(r0, ROWS_PER_TILE)], ss.at[0])
        init_desc.start()

        def in_descs(blk, p):
            b0 = slab + blk * BLK
            return (pltpu.make_async_copy(gsrc.at[pl.ds(b0, BLK)], gin[p],
                                          sin.at[p]),
                    pltpu.make_async_copy(dst.at[pl.ds(b0, BLK)], din[p],
                                          sin.at[p]))
        for d in in_descs(0, 0):
            d.start()
        init_desc.wait()
        plsc.subcore_barrier()

        def compact16(k, off_v, p):
            g16 = gin[p][pl.ds(k * 16, 16)]
            d16 = din[p][pl.ds(k * 16, 16)]
            ld = d16 - lo
            mask = ld.astype(jnp.uint32) < jnp.uint32(CHUNK)
            packed = lax.shift_left(g16, SHIFT) | ld
            cum = jnp.cumsum(mask.astype(jnp.int32))
            plsc.store_scatter(pairs, [(off_v + (cum - 1)) & (CAP_P - 1)],
                               packed, mask=mask)
            return off_v + plsc.all_reduce_population_count(mask)

        def block_pair(b2, state):
            off_v, fb = state
            for p in (0, 1):
                blk = b2 * 2 + p
                for d in in_descs(blk, p):
                    d.wait()

                @pl.when(blk + 1 < NBLK)
                def _():
                    for d in in_descs(blk + 1, 1 - p):
                        d.start()

                off_v = lax.fori_loop(
                    0, BLK // 16, lambda k, o: compact16(k, o, p), off_v)
                navail = lax.shift_right_logical(off_v[0], 7)
                lax.fori_loop(fb, navail, fire, jnp.int32(0))
                fb = navail
            return off_v, fb

        off_v, fb = lax.fori_loop(
            0, NBLK // 2, block_pair,
            (jnp.zeros((16,), jnp.int32), jnp.int32(0)))

        off_s = off_v[0]
        base = off_s & (CAP_P - 1)
        for i in range(BATCH // 16):
            pairs[pl.ds(base + i * 16, 16)] = junk16
        nbat = lax.shift_right_logical(off_s + (BATCH - 1), 7)
        lax.fori_loop(fb, nbat, fire, jnp.int32(0))
        lax.fori_loop(lax.max(nbat - LAG, 0), nbat, chase, jnp.int32(0))
        lax.fori_loop(lax.max(nbat - RING, 0), nbat, drain_scatter,
                      jnp.int32(0))
        plsc.subcore_barrier()

        @pl.when(tid < NS - 1)
        def _():
            pltpu.sync_copy(acc.at[pl.ds(r0, ROWS_PER_TILE)],
                            out.at[pl.ds(lo + r0, ROWS_PER_TILE)])

        last0 = (NS - 1) * ROWS_PER_TILE
        nfull = CHUNK - last0
        if j < NCH // 2 - 1:
            @pl.when(tid == NS - 1)
            def _():
                pltpu.sync_copy(acc.at[pl.ds(last0, nfull)],
                                out.at[pl.ds(lo + last0, nfull)])
        else:
            nshort = N_NODES - (NCH - 1) * CHUNK - last0

            @pl.when((tid == NS - 1) & (cid == 0))
            def _():
                pltpu.sync_copy(acc.at[pl.ds(last0, nfull)],
                                out.at[pl.ds(lo + last0, nfull)])

            @pl.when((tid == NS - 1) & (cid == 1))
            def _():
                pltpu.sync_copy(acc.at[pl.ds(last0, nshort)],
                                out.at[pl.ds(lo + last0, nshort)])

    for j in range(NCH // 2):
        do_chunk(j)


def _sc_aggregate(tab, gsrc, dst):
    mesh = plsc.VectorSubcoreMesh(core_axis_name="c", subcore_axis_name="s")
    scratch = (
        [pltpu.VMEM_SHARED((ACC_ROWS, D), jnp.float32),
         pltpu.VMEM((CAP_P + BATCH,), jnp.int32)]
        + [pltpu.VMEM((BLK,), jnp.int32) for _ in range(4)]
        + [pltpu.VMEM((RING, BATCH), jnp.int32) for _ in range(2)]
        + [pltpu.VMEM((RING, BATCH, D), jnp.float32),
           pltpu.SemaphoreType.DMA((2,)),
           pltpu.SemaphoreType.DMA((RING,)),
           pltpu.SemaphoreType.DMA((RING,))]
    )
    fn = pl.kernel(
        _sc_body,
        out_type=jax.ShapeDtypeStruct((N_NODES, D), jnp.float32),
        mesh=mesh,
        scratch_types=scratch,
        compiler_params=pltpu.CompilerParams(needs_layout_passes=False),
    )
    return fn(tab, gsrc, dst)


def kernel(x, edge_index_0, edge_index_1, edge_index_2, edge_index_3,
           W_root, b_root, W_conv):
    w_cat = jnp.concatenate([W_root[None], W_conv], axis=0)
    tab = _tc_table(x, w_cat, b_root.reshape(1, D))

    eis = (edge_index_0, edge_index_1, edge_index_2, edge_index_3)
    pad = E_PAD - E_TOT
    gsrc = jnp.concatenate(
        [ei[0].astype(jnp.int32) + l * N_NODES for l, ei in enumerate(eis)]
        + [jnp.full((pad,), PAD_SRC, jnp.int32)])
    dst = jnp.concatenate(
        [ei[1].astype(jnp.int32) for ei in eis]
        + [jnp.full((pad,), 2 * N_NODES, jnp.int32)])

    return _sc_aggregate(tab, gsrc, dst)


# speedup vs baseline: 1.2656x; 1.2656x over previous
import jax
import jax.numpy as jnp
from jax import lax
from jax.experimental import pallas as pl
from jax.experimental.pallas import tpu as pltpu
from jax.experimental.pallas import tpu_sc as plsc

N_NODES = 50000
D = 128
N_LABELS = 4
E_PER_LABEL = 150000

NS = 16
E_TOT = N_LABELS * E_PER_LABEL
BLK = 1024
NBLK = 40
EPT = NBLK * BLK
E_PAD = NS * EPT
BATCH = 64
RING = 8
LAG = 4
NCH = 8
CHUNK = 6256
ROWS_PER_TILE = 392
ACC_ROWS = 6272
SHIFT = 13
LMASK = (1 << SHIFT) - 1
JUNK_LDST = 6260
PAD_SRC = 0
CAP_P = 8192
TAB_ROWS = (N_LABELS + 1) * N_NODES
RB = 2000


def _tc_body(x_ref, w_ref, b_ref, o_ref):
    j = pl.program_id(1)
    y = jnp.dot(x_ref[...], w_ref[0], preferred_element_type=jnp.float32)
    o_ref[...] = y + jnp.where(j == 0, b_ref[...], jnp.zeros_like(b_ref[...]))


def _tc_table(x, w_cat, b2d):
    n_i = N_NODES // RB
    return pl.pallas_call(
        _tc_body,
        grid=(n_i, N_LABELS + 1),
        in_specs=[
            pl.BlockSpec((RB, D), lambda i, j: (i, 0)),
            pl.BlockSpec((1, D, D), lambda i, j: (j, 0, 0)),
            pl.BlockSpec((1, D), lambda i, j: (0, 0)),
        ],
        out_specs=pl.BlockSpec((RB, D), lambda i, j: (j * (N_NODES // RB) + i, 0)),
        out_shape=jax.ShapeDtypeStruct((TAB_ROWS, D), jnp.float32),
    )(x, w_cat, b2d)


def _sc_body(tab, gsrc, dst, out,
             acc, pairs, gin0, gin1, din0, din1, gq, dq, rows, sin, sg, ss):
    gin = (gin0, gin1)
    din = (din0, din1)
    cid = lax.axis_index("c")
    tid = lax.axis_index("s")
    slab = tid * EPT
    r0 = tid * ROWS_PER_TILE
    junk16 = jnp.full((16,), JUNK_LDST, jnp.int32)

    def wait_gather(s):
        pltpu.make_async_copy(tab.at[gq.at[s]], rows.at[s], sg.at[s]).wait()

    def wait_scatter(s):
        pltpu.make_async_copy(rows.at[s], acc.at[dq.at[s]], ss.at[s]).wait()

    def fire(b, carry):
        s = lax.rem(b, RING)

        @pl.when(b >= RING)
        def _():
            wait_scatter(s)
        base = (b * BATCH) & (CAP_P - 1)
        for k in range(BATCH // 16):
            p16 = pairs[pl.ds(base + k * 16, 16)]
            gq[s, pl.ds(k * 16, 16)] = (
                lax.shift_right_logical(p16, SHIFT) + N_NODES)
            dq[s, pl.ds(k * 16, 16)] = p16 & LMASK
        pltpu.async_copy(tab.at[gq.at[s]], rows.at[s], sg.at[s])

        @pl.when(b >= LAG)
        def _():
            s2 = lax.rem(b - LAG, RING)
            wait_gather(s2)
            pltpu.async_copy(rows.at[s2], acc.at[dq.at[s2]], ss.at[s2],
                             add=True)
        return carry

    def chase(t, carry):
        s2 = lax.rem(t, RING)
        wait_gather(s2)
        pltpu.async_copy(rows.at[s2], acc.at[dq.at[s2]], ss.at[s2], add=True)
        return carry

    def drain_scatter(t, carry):
        wait_scatter(lax.rem(t, RING))
        return carry

    def do_chunk(j):
        lo = (cid * (NCH // 2) + j) * CHUNK

        init_desc = pltpu.make_async_copy(
            tab.at[pl.ds(lo + r0, ROWS_PER_TILE)],
            acc.at[pl.ds(r0, ROWS_PER_TILE)], ss.at[0])
        init_desc.start()

        def in_descs(blk, p):
            b0 = slab + blk * BLK
            return (pltpu.make_async_copy(gsrc.at[pl.ds(b0, BLK)], gin[p],
                                          sin.at[p]),
                    pltpu.make_async_copy(dst.at[pl.ds(b0, BLK)], din[p],
                                          sin.at[p]))
        for d in in_descs(0, 0):
            d.start()
        init_desc.wait()
        plsc.subcore_barrier()

        def compact16(k, off_v, p):
            g16 = gin[p][pl.ds(k * 16, 16)]
            d16 = din[p][pl.ds(k * 16, 16)]
            ld = d16 - lo
            mask = ld.astype(jnp.uint32) < jnp.uint32(CHUNK)
            packed = lax.shift_left(g16, SHIFT) | ld
            cum = jnp.cumsum(mask.astype(jnp.int32))
            plsc.store_scatter(pairs, [(off_v + (cum - 1)) & (CAP_P - 1)],
                               packed, mask=mask)
            return off_v + plsc.all_reduce_population_count(mask)

        def block_pair(b2, state):
            off_v, fb = state
            for p in (0, 1):
                blk = b2 * 2 + p
                for d in in_descs(blk, p):
                    d.wait()

                @pl.when(blk + 1 < NBLK)
                def _():
                    for d in in_descs(blk + 1, 1 - p):
                        d.start()

                off_v = lax.fori_loop(
                    0, BLK // 16, lambda k, o: compact16(k, o, p), off_v)
                navail = lax.shift_right_logical(off_v[0], 6)
                lax.fori_loop(fb, navail, fire, jnp.int32(0))
                fb = navail
            return off_v, fb

        off_v, fb = lax.fori_loop(
            0, NBLK // 2, block_pair,
            (jnp.zeros((16,), jnp.int32), jnp.int32(0)))

        off_s = off_v[0]
        base = off_s & (CAP_P - 1)
        for i in range(BATCH // 16):
            pairs[pl.ds(base + i * 16, 16)] = junk16
        nbat = lax.shift_right_logical(off_s + (BATCH - 1), 6)
        lax.fori_loop(fb, nbat, fire, jnp.int32(0))
        lax.fori_loop(lax.max(nbat - LAG, 0), nbat, chase, jnp.int32(0))
        lax.fori_loop(lax.max(nbat - RING, 0), nbat, drain_scatter,
                      jnp.int32(0))
        plsc.subcore_barrier()

        @pl.when(tid < NS - 1)
        def _():
            pltpu.sync_copy(acc.at[pl.ds(r0, ROWS_PER_TILE)],
                            out.at[pl.ds(lo + r0, ROWS_PER_TILE)])

        last0 = (NS - 1) * ROWS_PER_TILE
        nfull = CHUNK - last0
        if j < NCH // 2 - 1:
            @pl.when(tid == NS - 1)
            def _():
                pltpu.sync_copy(acc.at[pl.ds(last0, nfull)],
                                out.at[pl.ds(lo + last0, nfull)])
        else:
            nshort = N_NODES - (NCH - 1) * CHUNK - last0

            @pl.when((tid == NS - 1) & (cid == 0))
            def _():
                pltpu.sync_copy(acc.at[pl.ds(last0, nfull)],
                                out.at[pl.ds(lo + last0, nfull)])

            @pl.when((tid == NS - 1) & (cid == 1))
            def _():
                pltpu.sync_copy(acc.at[pl.ds(last0, nshort)],
                                out.at[pl.ds(lo + last0, nshort)])

    for j in range(NCH // 2):
        do_chunk(j)


def _sc_aggregate(tab, gsrc, dst):
    mesh = plsc.VectorSubcoreMesh(core_axis_name="c", subcore_axis_name="s")
    scratch = (
        [pltpu.VMEM_SHARED((ACC_ROWS, D), jnp.float32),
         pltpu.VMEM((CAP_P + BATCH,), jnp.int32)]
        + [pltpu.VMEM((BLK,), jnp.int32) for _ in range(4)]
        + [pltpu.VMEM((RING, BATCH), jnp.int32) for _ in range(2)]
        + [pltpu.VMEM((RING, BATCH, D), jnp.float32),
           pltpu.SemaphoreType.DMA((2,)),
           pltpu.SemaphoreType.DMA((RING,)),
           pltpu.SemaphoreType.DMA((RING,))]
    )
    fn = pl.kernel(
        _sc_body,
        out_type=jax.ShapeDtypeStruct((N_NODES, D), jnp.float32),
        mesh=mesh,
        scratch_types=scratch,
        compiler_params=pltpu.CompilerParams(needs_layout_passes=False),
    )
    return fn(tab, gsrc, dst)


def kernel(x, edge_index_0, edge_index_1, edge_index_2, edge_index_3,
           W_root, b_root, W_conv):
    w_cat = jnp.concatenate([W_root[None], W_conv], axis=0)
    tab = _tc_table(x, w_cat, b_root.reshape(1, D))

    eis = (edge_index_0, edge_index_1, edge_index_2, edge_index_3)
    pad = E_PAD - E_TOT
    gsrc = jnp.concatenate(
        [ei[0].astype(jnp.int32) + l * N_NODES for l, ei in enumerate(eis)]
        + [jnp.full((pad,), PAD_SRC, jnp.int32)])
    dst = jnp.concatenate(
        [ei[1].astype(jnp.int32) for ei in eis]
        + [jnp.full((pad,), 2 * N_NODES, jnp.int32)])

    return _sc_aggregate(tab, gsrc, dst)
